# baseline (speedup 1.0000x reference)
import functools
import math

import jax
import jax.numpy as jnp
import numpy as np
from jax import lax
from jax.experimental import pallas as pl
from jax.experimental.pallas import tpu as pltpu
from jax.experimental.pallas import tpu_sc as plsc

D_MODEL = 64
MAX_LEN = 5000
BATCH = 4096
SEQ = 200
SCALE = math.sqrt(D_MODEL)
OUT_D = 128

NC, NS = 2, 16
NW = NC * NS
SEQ_PER_W = BATCH // NW

CHUNK = 100
CHUNKS_PER_SEQ = SEQ // CHUNK
CHUNKS_PER_W = SEQ_PER_W * CHUNKS_PER_SEQ

NGBUF = 4
NOBUF = 2
LOOKAHEAD = 2


def _make_pe():
    pe = np.zeros((MAX_LEN, D_MODEL), dtype=np.float32)
    pos = np.arange(MAX_LEN, dtype=np.float32)[:, None]
    div_term = np.exp(
        np.arange(0, D_MODEL, 2, dtype=np.float32) * (-math.log(10000.0) / D_MODEL)
    )
    pe[:, 0::2] = np.sin(pos * div_term)
    pe[:, 1::2] = np.cos(pos * div_term)
    return jnp.asarray(pe[:SEQ])


_mesh = plsc.VectorSubcoreMesh(core_axis_name="c", subcore_axis_name="s")


@functools.partial(
    pl.kernel,
    out_type=jax.ShapeDtypeStruct((BATCH, SEQ, OUT_D), jnp.float32),
    mesh=_mesh,
    scratch_types=[
        pltpu.VMEM((CHUNKS_PER_W, CHUNK), jnp.int32),
        pltpu.VMEM((SEQ, D_MODEL), jnp.float32),
        [pltpu.VMEM((CHUNK, D_MODEL), jnp.float32) for _ in range(NGBUF)],
        [pltpu.VMEM((CHUNK, OUT_D), jnp.float32) for _ in range(NOBUF)],
        [pltpu.SemaphoreType.DMA for _ in range(NGBUF)],
        [pltpu.SemaphoreType.DMA for _ in range(NOBUF)],
    ],
    compiler_params=pltpu.CompilerParams(use_tc_tiling_on_sc=False),
)
def _sc_kernel(x_hbm, pe_hbm, table_hbm, out_hbm, idx_v, pe_v, gbuf, obuf, gsem, osem):
    wid = lax.axis_index("s") * NC + lax.axis_index("c")
    idx_base = wid * CHUNKS_PER_W
    batch_base = wid * SEQ_PER_W

    pltpu.sync_copy(x_hbm.at[pl.ds(idx_base, CHUNKS_PER_W)], idx_v)
    pltpu.sync_copy(pe_hbm, pe_v)

    def issue_gather(t, g):
        pltpu.async_copy(table_hbm.at[idx_v.at[t]], gbuf[g], gsem[g])

    def wait_gather(g):
        pltpu.make_async_copy(table_hbm.at[pl.ds(0, CHUNK)], gbuf[g], gsem[g]).wait()

    def wait_outcopy(o):
        pltpu.make_async_copy(
            obuf[o], out_hbm.at[0, pl.ds(0, CHUNK)], osem[o]
        ).wait()

    for g in range(LOOKAHEAD):
        issue_gather(g, g)

    def step(u, _):
        for g in range(NGBUF):
            t = u * NGBUF + g
            o = g % NOBUF
            pe_off = (g % CHUNKS_PER_SEQ) * CHUNK

            @pl.when(t + LOOKAHEAD < CHUNKS_PER_W)
            def _():
                issue_gather(t + LOOKAHEAD, (g + LOOKAHEAD) % NGBUF)

            @pl.when(t >= NOBUF)
            def _():
                wait_outcopy(o)

            wait_gather(g)

            def fma_row(r, _):
                for j in range(D_MODEL // 16):
                    sl = pl.ds(j * 16, 16)
                    obuf[o][r, sl] = gbuf[g][r, sl] * SCALE + pe_v[pe_off + r, sl]
                return ()

            lax.fori_loop(0, CHUNK, fma_row, ())
            pltpu.async_copy(
                obuf[o],
                out_hbm.at[batch_base + t // CHUNKS_PER_SEQ, pl.ds(pe_off, CHUNK)],
                osem[o],
            )
        return ()

    lax.fori_loop(0, CHUNKS_PER_W // NGBUF, step, ())
    for o in range(NOBUF):
        wait_outcopy(o)


def kernel(x, table):
    pe = _make_pe()
    x2 = x.reshape(CHUNKS_PER_W * NW, CHUNK)
    out = _sc_kernel(x2, pe, table)
    return out[:, :, :D_MODEL]


# speedup vs baseline: 1.0008x; 1.0008x over previous
import functools
import math

import jax
import jax.numpy as jnp
import numpy as np
from jax import lax
from jax.experimental import pallas as pl
from jax.experimental.pallas import tpu as pltpu
from jax.experimental.pallas import tpu_sc as plsc

D_MODEL = 64
MAX_LEN = 5000
BATCH = 4096
SEQ = 200
SCALE = math.sqrt(D_MODEL)
OUT_D = 128

NC, NS = 2, 16
NW = NC * NS
SEQ_PER_W = BATCH // NW

IDX_COLS = 100
IDX_ROWS_PER_SEQ = SEQ // IDX_COLS

NGBUF = 3
NOBUF = 2
LOOKAHEAD = 2


def _make_pe():
    pe = np.zeros((MAX_LEN, D_MODEL), dtype=np.float32)
    pos = np.arange(MAX_LEN, dtype=np.float32)[:, None]
    div_term = np.exp(
        np.arange(0, D_MODEL, 2, dtype=np.float32) * (-math.log(10000.0) / D_MODEL)
    )
    pe[:, 0::2] = np.sin(pos * div_term)
    pe[:, 1::2] = np.cos(pos * div_term)
    return jnp.asarray(pe[:SEQ])


_mesh = plsc.VectorSubcoreMesh(core_axis_name="c", subcore_axis_name="s")


@functools.partial(
    pl.kernel,
    out_type=jax.ShapeDtypeStruct((BATCH, SEQ, OUT_D), jnp.float32),
    mesh=_mesh,
    scratch_types=[
        pltpu.VMEM((SEQ_PER_W * IDX_ROWS_PER_SEQ, IDX_COLS), jnp.int32),
        pltpu.VMEM((SEQ, D_MODEL), jnp.float32),
        [pltpu.VMEM((SEQ, D_MODEL), jnp.float32) for _ in range(NGBUF)],
        [pltpu.VMEM((SEQ, OUT_D), jnp.float32) for _ in range(NOBUF)],
        [pltpu.SemaphoreType.DMA for _ in range(NGBUF)],
        [pltpu.SemaphoreType.DMA for _ in range(NOBUF)],
    ],
    compiler_params=pltpu.CompilerParams(use_tc_tiling_on_sc=False),
)
def _sc_kernel(x_hbm, pe_hbm, table_hbm, out_hbm, idx_v, pe_v, gbuf, obuf, gsem, osem):
    wid = lax.axis_index("s") * NC + lax.axis_index("c")
    idx_base = wid * SEQ_PER_W * IDX_ROWS_PER_SEQ
    batch_base = wid * SEQ_PER_W

    pltpu.sync_copy(x_hbm.at[pl.ds(idx_base, SEQ_PER_W * IDX_ROWS_PER_SEQ)], idx_v)
    pltpu.sync_copy(pe_hbm, pe_v)

    def issue_gather(t, g):
        pltpu.async_copy(
            table_hbm.at[idx_v.at[2 * t]], gbuf[g].at[pl.ds(0, IDX_COLS)], gsem[g]
        )
        pltpu.async_copy(
            table_hbm.at[idx_v.at[2 * t + 1]],
            gbuf[g].at[pl.ds(IDX_COLS, IDX_COLS)],
            gsem[g],
        )

    def wait_gather(g):
        pltpu.make_async_copy(table_hbm.at[pl.ds(0, SEQ)], gbuf[g], gsem[g]).wait()

    def wait_outcopy(o):
        pltpu.make_async_copy(obuf[o], out_hbm.at[0], osem[o]).wait()

    for g in range(LOOKAHEAD):
        issue_gather(g, g)

    def step6(u, _):
        for k in range(6):
            t = u * 6 + k
            g = k % NGBUF
            o = k % NOBUF

            @pl.when(t + LOOKAHEAD < SEQ_PER_W)
            def _():
                issue_gather(t + LOOKAHEAD, (g + LOOKAHEAD) % NGBUF)

            @pl.when(t >= NOBUF)
            def _():
                wait_outcopy(o)

            wait_gather(g)

            def fma_row(r, _):
                for j in range(D_MODEL // 16):
                    sl = pl.ds(j * 16, 16)
                    obuf[o][r, sl] = gbuf[g][r, sl] * SCALE + pe_v[r, sl]
                return ()

            lax.fori_loop(0, SEQ, fma_row, ())
            pltpu.async_copy(obuf[o], out_hbm.at[batch_base + t], osem[o])
        return ()

    lax.fori_loop(0, SEQ_PER_W // 6, step6, ())
    for k in range(SEQ_PER_W - SEQ_PER_W % 6, SEQ_PER_W):
        t = k
        g = k % NGBUF
        o = k % NOBUF

        @pl.when(t >= NOBUF)
        def _():
            wait_outcopy(o)

        wait_gather(g)

        def fma_row(r, _):
            for j in range(D_MODEL // 16):
                sl = pl.ds(j * 16, 16)
                obuf[o][r, sl] = gbuf[g][r, sl] * SCALE + pe_v[r, sl]
            return ()

        lax.fori_loop(0, SEQ, fma_row, ())
        pltpu.async_copy(obuf[o], out_hbm.at[batch_base + t], osem[o])
    for o in range(NOBUF):
        wait_outcopy(o)


def kernel(x, table):
    pe = _make_pe()
    x2 = x.reshape(2 * BATCH, IDX_COLS)
    out = _sc_kernel(x2, pe, table)
    return out[:, :, :D_MODEL]


# speedup vs baseline: 1.7371x; 1.7357x over previous
import functools
import math

import jax
import jax.numpy as jnp
import numpy as np
from jax import lax
from jax.experimental import pallas as pl
from jax.experimental.pallas import tpu as pltpu
from jax.experimental.pallas import tpu_sc as plsc

D_MODEL = 64
MAX_LEN = 5000
BATCH = 4096
SEQ = 200
SCALE = math.sqrt(D_MODEL)
OUT_D = 128

NC, NS = 2, 16
NW = NC * NS
SEQ_PER_W = BATCH // NW

IDX_COLS = 100
IDX_ROWS_PER_SEQ = SEQ // IDX_COLS

NGBUF = 3
NOBUF = 2
LOOKAHEAD = 2


def _make_pe():
    pe = np.zeros((MAX_LEN, D_MODEL), dtype=np.float32)
    pos = np.arange(MAX_LEN, dtype=np.float32)[:, None]
    div_term = np.exp(
        np.arange(0, D_MODEL, 2, dtype=np.float32) * (-math.log(10000.0) / D_MODEL)
    )
    pe[:, 0::2] = np.sin(pos * div_term)
    pe[:, 1::2] = np.cos(pos * div_term)
    return jnp.asarray(pe[:SEQ])


_mesh = plsc.VectorSubcoreMesh(core_axis_name="c", subcore_axis_name="s")


@functools.partial(
    pl.kernel,
    out_type=jax.ShapeDtypeStruct((BATCH, SEQ, OUT_D), jnp.float32),
    mesh=_mesh,
    scratch_types=[
        pltpu.VMEM((SEQ_PER_W * IDX_ROWS_PER_SEQ, IDX_COLS), jnp.int32),
        pltpu.VMEM((SEQ, D_MODEL), jnp.float32),
        [pltpu.VMEM((SEQ, D_MODEL), jnp.float32) for _ in range(NGBUF)],
        [pltpu.VMEM((SEQ, OUT_D), jnp.float32) for _ in range(NOBUF)],
        [pltpu.SemaphoreType.DMA for _ in range(NGBUF)],
        [pltpu.SemaphoreType.DMA for _ in range(NOBUF)],
    ],
    compiler_params=pltpu.CompilerParams(use_tc_tiling_on_sc=False),
)
def _sc_kernel(x_hbm, pe_hbm, table_hbm, out_hbm, idx_v, pe_v, gbuf, obuf, gsem, osem):
    wid = lax.axis_index("s") * NC + lax.axis_index("c")
    idx_base = wid * SEQ_PER_W * IDX_ROWS_PER_SEQ
    batch_base = wid * SEQ_PER_W

    pltpu.sync_copy(x_hbm.at[pl.ds(idx_base, SEQ_PER_W * IDX_ROWS_PER_SEQ)], idx_v)
    pltpu.sync_copy(pe_hbm, pe_v)

    def issue_gather(t, g):
        pltpu.async_copy(
            table_hbm.at[idx_v.at[2 * t]], gbuf[g].at[pl.ds(0, IDX_COLS)], gsem[g]
        )
        pltpu.async_copy(
            table_hbm.at[idx_v.at[2 * t + 1]],
            gbuf[g].at[pl.ds(IDX_COLS, IDX_COLS)],
            gsem[g],
        )

    def wait_gather(g):
        pltpu.make_async_copy(table_hbm.at[pl.ds(0, SEQ)], gbuf[g], gsem[g]).wait()

    def wait_outcopy(o):
        pltpu.make_async_copy(obuf[o], out_hbm.at[0], osem[o]).wait()

    def do_fma(g, o):
        @plsc.parallel_loop(0, SEQ, unroll=8)
        def _(r):
            for j in range(D_MODEL // 16):
                sl = pl.ds(j * 16, 16)
                obuf[o][r, sl] = gbuf[g][r, sl] * SCALE + pe_v[r, sl]

    for g in range(LOOKAHEAD):
        issue_gather(g, g)

    def step6(u, _):
        for k in range(6):
            t = u * 6 + k
            g = k % NGBUF
            o = k % NOBUF

            @pl.when(t + LOOKAHEAD < SEQ_PER_W)
            def _():
                issue_gather(t + LOOKAHEAD, (g + LOOKAHEAD) % NGBUF)

            @pl.when(t >= NOBUF)
            def _():
                wait_outcopy(o)

            wait_gather(g)
            do_fma(g, o)
            pltpu.async_copy(obuf[o], out_hbm.at[batch_base + t], osem[o])
        return ()

    lax.fori_loop(0, SEQ_PER_W // 6, step6, ())
    for k in range(SEQ_PER_W - SEQ_PER_W % 6, SEQ_PER_W):
        t = k
        g = k % NGBUF
        o = k % NOBUF

        @pl.when(t >= NOBUF)
        def _():
            wait_outcopy(o)

        wait_gather(g)
        do_fma(g, o)
        pltpu.async_copy(obuf[o], out_hbm.at[batch_base + t], osem[o])
    for o in range(NOBUF):
        wait_outcopy(o)


def kernel(x, table):
    pe = _make_pe()
    x2 = x.reshape(2 * BATCH, IDX_COLS)
    out = _sc_kernel(x2, pe, table)
    return out[:, :, :D_MODEL]


# speedup vs baseline: 1.7429x; 1.0033x over previous
import functools
import math

import jax
import jax.numpy as jnp
import numpy as np
from jax import lax
from jax.experimental import pallas as pl
from jax.experimental.pallas import tpu as pltpu
from jax.experimental.pallas import tpu_sc as plsc

D_MODEL = 64
MAX_LEN = 5000
BATCH = 4096
SEQ = 200
SCALE = math.sqrt(D_MODEL)
OUT_D = 128

NC, NS = 2, 16
NW = NC * NS
SEQ_PER_W = BATCH // NW

IDX_COLS = 100
IDX_ROWS_PER_SEQ = SEQ // IDX_COLS

NGBUF = 3
NOBUF = 2
LOOKAHEAD = 2


def _make_pe():
    pe = np.zeros((MAX_LEN, D_MODEL), dtype=np.float32)
    pos = np.arange(MAX_LEN, dtype=np.float32)[:, None]
    div_term = np.exp(
        np.arange(0, D_MODEL, 2, dtype=np.float32) * (-math.log(10000.0) / D_MODEL)
    )
    pe[:, 0::2] = np.sin(pos * div_term)
    pe[:, 1::2] = np.cos(pos * div_term)
    return jnp.asarray(pe[:SEQ])


_mesh = plsc.VectorSubcoreMesh(core_axis_name="c", subcore_axis_name="s")


@functools.partial(
    pl.kernel,
    out_type=jax.ShapeDtypeStruct((BATCH, SEQ, OUT_D), jnp.float32),
    mesh=_mesh,
    scratch_types=[
        pltpu.VMEM((SEQ_PER_W * IDX_ROWS_PER_SEQ, IDX_COLS), jnp.int32),
        pltpu.VMEM((SEQ, D_MODEL), jnp.float32),
        [pltpu.VMEM((SEQ, D_MODEL), jnp.float32) for _ in range(NGBUF)],
        [pltpu.VMEM((SEQ, OUT_D), jnp.float32) for _ in range(NOBUF)],
        [pltpu.SemaphoreType.DMA for _ in range(NGBUF)],
        [pltpu.SemaphoreType.DMA for _ in range(NOBUF)],
    ],
    compiler_params=pltpu.CompilerParams(use_tc_tiling_on_sc=False),
)
def _sc_kernel(x_hbm, pe_hbm, table_hbm, out_hbm, idx_v, pe_v, gbuf, obuf, gsem, osem):
    wid = lax.axis_index("s") * NC + lax.axis_index("c")
    idx_base = wid * SEQ_PER_W * IDX_ROWS_PER_SEQ
    batch_base = wid * SEQ_PER_W

    pltpu.sync_copy(x_hbm.at[pl.ds(idx_base, SEQ_PER_W * IDX_ROWS_PER_SEQ)], idx_v)
    pltpu.sync_copy(pe_hbm, pe_v)

    def issue_gather(t, g):
        pltpu.async_copy(
            table_hbm.at[idx_v.at[2 * t]], gbuf[g].at[pl.ds(0, IDX_COLS)], gsem[g]
        )
        pltpu.async_copy(
            table_hbm.at[idx_v.at[2 * t + 1]],
            gbuf[g].at[pl.ds(IDX_COLS, IDX_COLS)],
            gsem[g],
        )

    def wait_gather(g):
        pltpu.make_async_copy(table_hbm.at[pl.ds(0, SEQ)], gbuf[g], gsem[g]).wait()

    def wait_outcopy(o):
        pltpu.make_async_copy(obuf[o], out_hbm.at[0], osem[o]).wait()

    def do_fma(g, o):
        @plsc.parallel_loop(0, SEQ, unroll=8)
        def _(r):
            for j in range(D_MODEL // 16):
                sl = pl.ds(j * 16, 16)
                obuf[o][r, sl] = gbuf[g][r, sl] * SCALE + pe_v[r, sl]

    for g in range(LOOKAHEAD):
        issue_gather(g, g)

    def step6(u, _):
        for k in range(6):
            t = u * 6 + k
            g = k % NGBUF
            o = k % NOBUF

            @pl.when(t + LOOKAHEAD < SEQ_PER_W)
            def _():
                issue_gather(t + LOOKAHEAD, (g + LOOKAHEAD) % NGBUF)

            @pl.when(t >= NOBUF)
            def _():
                wait_outcopy(o)

            wait_gather(g)
            do_fma(g, o)
            pltpu.async_copy(obuf[o], out_hbm.at[batch_base + t], osem[o])
        return ()

    lax.fori_loop(0, SEQ_PER_W // 6, step6, ())
    for k in range(SEQ_PER_W - SEQ_PER_W % 6, SEQ_PER_W):
        t = k
        g = k % NGBUF
        o = k % NOBUF

        @pl.when(t >= NOBUF)
        def _():
            wait_outcopy(o)

        wait_gather(g)
        do_fma(g, o)
        pltpu.async_copy(obuf[o], out_hbm.at[batch_base + t], osem[o])
    for o in range(NOBUF):
        wait_outcopy(o)


def kernel(x, table):
    pe = _make_pe()
    x2 = x.reshape(2 * BATCH, IDX_COLS)
    table_lin = lax.optimization_barrier(table.reshape(-1)).reshape(table.shape)
    out = _sc_kernel(x2, pe, table_lin)
    return out[:, :, :D_MODEL]
